# baseline (speedup 1.0000x reference)
import functools

import jax
import jax.numpy as jnp
from jax import lax
from jax.experimental import pallas as pl
from jax.experimental.pallas import tpu as pltpu
from jax.experimental.pallas import tpu_sc as plsc

NUM_FIELDS = 26
VOCAB = 100000
EMB_DIM = 16
DENSE_DIM = 13
BATCH = 4096
H1, H2 = 64, 32
BN_EPS = 1e-5

FM_COLS = NUM_FIELDS * EMB_DIM

_NC, _NS = 2, 16
_NW = _NC * _NS
_BF = BATCH * NUM_FIELDS
_ROWS_PER_W = _BF // _NW


def _sc_gather(fm_tab_flat, lin_tab_flat, idx_flat):
    mesh = plsc.VectorSubcoreMesh(core_axis_name="c", subcore_axis_name="s")

    @functools.partial(
        pl.kernel,
        mesh=mesh,
        compiler_params=pltpu.CompilerParams(use_tc_tiling_on_sc=False),
        out_type=(
            jax.ShapeDtypeStruct((_BF, EMB_DIM), jnp.float32),
            jax.ShapeDtypeStruct((_BF,), jnp.float32),
        ),
        scratch_types=[
            pltpu.VMEM((_ROWS_PER_W,), jnp.int32),
            pltpu.VMEM((_ROWS_PER_W, EMB_DIM), jnp.float32),
            pltpu.VMEM((_ROWS_PER_W,), jnp.float32),
            pltpu.SemaphoreType.DMA,
            pltpu.SemaphoreType.DMA,
        ],
    )
    def k(fm_hbm, lin_hbm, idx_hbm, fm_out, lin_out, idx_v, rows_v, lin_v,
          sem_a, sem_b):
        wid = lax.axis_index("s") * _NC + lax.axis_index("c")
        base = wid * _ROWS_PER_W
        pltpu.sync_copy(idx_hbm.at[pl.ds(base, _ROWS_PER_W)], idx_v)
        cp_a = pltpu.async_copy(fm_hbm.at[idx_v], rows_v, sem_a)
        cp_b = pltpu.async_copy(lin_hbm.at[idx_v], lin_v, sem_b)
        cp_a.wait()
        cp_b.wait()
        pltpu.sync_copy(rows_v, fm_out.at[pl.ds(base, _ROWS_PER_W)])
        pltpu.sync_copy(lin_v, lin_out.at[pl.ds(base, _ROWS_PER_W)])

    return k(fm_tab_flat, lin_tab_flat, idx_flat)


def _tc_body(fm_ref, dense_ref, lin_ref, sel_ref, wd_ref, w1_ref, b1_ref,
             g1_ref, bt1_ref, w2_ref, b2_ref, g2_ref, bt2_ref, wout_ref,
             cbias_ref, out_ref):
    f32 = jnp.float32
    x = fm_ref[...]
    d = dense_ref[...]
    sel = sel_ref[...]
    dn = (((1,), (1,)), ((), ()))

    sv = jnp.dot(x, sel, preferred_element_type=f32)
    sq = jnp.dot(x * x, sel, preferred_element_type=f32)
    fm_logit = 0.5 * jnp.sum(sv * sv - sq, axis=1, keepdims=True)

    lin_logit = jnp.sum(lin_ref[...], axis=1, keepdims=True)
    lin_logit = lin_logit + lax.dot_general(d, wd_ref[...], dn,
                                            preferred_element_type=f32)

    inv = lax.rsqrt(jnp.float32(1.0 + BN_EPS))
    w1 = w1_ref[...]
    z = lax.dot_general(x, w1[:, :FM_COLS], dn, preferred_element_type=f32)
    z = z + lax.dot_general(d, w1[:, FM_COLS:], dn, preferred_element_type=f32)
    h = jnp.maximum((z + b1_ref[...]) * (g1_ref[...] * inv) + bt1_ref[...], 0.0)
    z2 = lax.dot_general(h, w2_ref[...], dn, preferred_element_type=f32)
    h2 = jnp.maximum((z2 + b2_ref[...]) * (g2_ref[...] * inv) + bt2_ref[...],
                     0.0)
    dnn_logit = lax.dot_general(h2, wout_ref[...], dn,
                                preferred_element_type=f32)

    total = lin_logit + fm_logit + dnn_logit + cbias_ref[...]
    out_ref[...] = jax.nn.sigmoid(total)


def _tc_dense(fm_emb, dense_inputs, lin_vals, sel, wd, w1, b1, g1, bt1, w2,
              b2, g2, bt2, wout, cbias):
    bm = 1024
    grid = (BATCH // bm,)
    full = lambda shape: pl.BlockSpec(shape, lambda i: (0,) * len(shape))
    row = lambda cols: pl.BlockSpec((bm, cols), lambda i: (i, 0))
    return pl.pallas_call(
        _tc_body,
        grid=grid,
        in_specs=[
            row(FM_COLS),
            row(DENSE_DIM),
            row(NUM_FIELDS),
            full((FM_COLS, EMB_DIM)),
            full((1, DENSE_DIM)),
            full((H1, FM_COLS + DENSE_DIM)),
            full((1, H1)), full((1, H1)), full((1, H1)),
            full((H2, H1)),
            full((1, H2)), full((1, H2)), full((1, H2)),
            full((1, H2)),
            full((1, 1)),
        ],
        out_specs=row(1),
        out_shape=jax.ShapeDtypeStruct((BATCH, 1), jnp.float32),
    )(fm_emb, dense_inputs, lin_vals, sel, wd, w1, b1, g1, bt1, w2, b2, g2,
      bt2, wout, cbias)


def kernel(sparse_inputs, dense_inputs, fm_tables, lin_tables, W_dense,
           b_dense, bias, W1, b1, g1, bt1, W2, b2, g2, bt2, Wout, bout):
    i32 = jnp.int32
    idx_flat = (sparse_inputs.astype(i32)
                + (jnp.arange(NUM_FIELDS, dtype=i32) * VOCAB)[None, :])
    idx_flat = idx_flat.reshape(-1)
    fm_flat = fm_tables.reshape(NUM_FIELDS * VOCAB, EMB_DIM)
    lin_flat = lin_tables.reshape(NUM_FIELDS * VOCAB)

    fm_rows, lin_vals = _sc_gather(fm_flat, lin_flat, idx_flat)
    fm_emb = fm_rows.reshape(BATCH, FM_COLS)
    lin_mat = lin_vals.reshape(BATCH, NUM_FIELDS)

    sel = jnp.tile(jnp.eye(EMB_DIM, dtype=jnp.float32), (NUM_FIELDS, 1))
    cbias = (bias + b_dense + bout).reshape(1, 1)
    out = _tc_dense(
        fm_emb, dense_inputs, lin_mat, sel, W_dense, W1,
        b1.reshape(1, H1), g1.reshape(1, H1), bt1.reshape(1, H1),
        W2, b2.reshape(1, H2), g2.reshape(1, H2), bt2.reshape(1, H2),
        Wout, cbias)
    return out.reshape(BATCH)
